# Initial kernel scaffold; baseline (speedup 1.0000x reference)
import jax
import jax.numpy as jnp
from jax.experimental import pallas as pl


def kernel(x, table):
    raise NotImplementedError("write your pallas kernel here")


# speedup vs baseline: 5.1739x; 5.1739x over previous
import jax
import jax.numpy as jnp
from jax import lax
from jax.experimental import pallas as pl
from jax.experimental.pallas import tpu as pltpu
from jax.experimental.pallas import tpu_sc as plsc

D = 64
NC, NS = 2, 16
NW = NC * NS
LW = 128
KIR = 8
SUBJ = 4
SUB = SUBJ * LW


def kernel(x, table):
    B0, H = x.shape
    B = B0 * H
    nrows = B // LW
    nblk = nrows // (KIR * NW)
    idx = x.reshape(nrows, LW).astype(jnp.int32)

    def body(idx_hbm, table_hbm, out_hbm, idx_v, rows_v, sem_a, sem_b):
        sems = (sem_a, sem_b)
        wid = lax.axis_index("s") * NC + lax.axis_index("c")
        blk0 = wid * nblk

        def load_idx(slot, blk):
            pltpu.sync_copy(idx_hbm.at[pl.ds(blk * KIR, KIR)], idx_v.at[slot])

        def fire(p, slot):
            for j in range(SUBJ):
                pltpu.async_copy(table_hbm.at[idx_v.at[slot, p * SUBJ + j]],
                                 rows_v.at[p, pl.ds(j * LW, LW)], sems[p])

        def drain_store(p, blk):
            for j in range(SUBJ):
                pltpu.make_async_copy(table_hbm.at[idx_v.at[0, j]],
                                      rows_v.at[p, pl.ds(j * LW, LW)],
                                      sems[p]).wait()
            pltpu.sync_copy(rows_v.at[p],
                            out_hbm.at[pl.ds((blk * 2 + p) * SUB, SUB)])

        load_idx(0, blk0)
        for p in range(2):
            fire(p, 0)

        @pl.loop(0, nblk, step=2)
        def _(m):
            blk = blk0 + m

            @pl.when(m + 1 < nblk)
            def _():
                load_idx(1, blk + 1)

            for p in range(2):
                drain_store(p, blk)

                @pl.when(m + 1 < nblk)
                def _():
                    fire(p, 1)

            @pl.when(m + 2 < nblk)
            def _():
                load_idx(0, blk + 2)

            for p in range(2):
                @pl.when(m + 1 < nblk)
                def _():
                    drain_store(p, blk + 1)

                @pl.when(m + 2 < nblk)
                def _():
                    fire(p, 0)

    fn = pl.kernel(
        body,
        out_type=jax.ShapeDtypeStruct((B, D), jnp.float32),
        mesh=plsc.VectorSubcoreMesh(core_axis_name="c", subcore_axis_name="s"),
        compiler_params=pltpu.CompilerParams(use_tc_tiling_on_sc=False),
        scratch_types=[
            pltpu.VMEM((2, KIR, LW), jnp.int32),
            pltpu.VMEM((2, SUB, D), jnp.float32),
            pltpu.SemaphoreType.DMA,
            pltpu.SemaphoreType.DMA,
        ],
    )
    return fn(idx, table).reshape(B0, H, D)


# speedup vs baseline: 20.3983x; 3.9425x over previous
import jax
import jax.numpy as jnp
from jax import lax
from jax.experimental import pallas as pl
from jax.experimental.pallas import tpu as pltpu
from jax.experimental.pallas import tpu_sc as plsc

D = 64
NC, NS = 2, 16
NW = NC * NS
BLK = 128
RBUF = 4
IBUF = 8


def kernel(x, table):
    B0, H = x.shape
    nblk = H * (B0 // BLK)
    npw = nblk // NW
    xt = jnp.transpose(x).astype(jnp.int32)

    def body(xt_hbm, table_hbm, out_hbm, idx_v, buf_v, bufT_v,
             isems, gsems, ssems):
        wid = lax.axis_index("s") * NC + lax.axis_index("c")
        f0 = wid * npw

        iota = lax.iota(jnp.int32, 16)
        iq = [iota + 16 * q for q in range(D // 16)]

        def hbb(f):
            return lax.shift_right_logical(f, 7), lax.bitwise_and(f, 127)

        def fire_idx(j, f):
            h, bb = hbb(f)
            pltpu.async_copy(xt_hbm.at[h, pl.ds(bb * BLK, BLK)], idx_v[j],
                             isems[j])

        def fire_gather(j, s):
            pltpu.make_async_copy(xt_hbm.at[0, pl.ds(0, BLK)], idx_v[j],
                                  isems[j]).wait()
            pltpu.async_copy(table_hbm.at[idx_v[j]], buf_v[s], gsems[s])

        def transpose(s):
            @plsc.parallel_loop(0, BLK, unroll=4)
            def _(b):
                idx_b = jnp.full((16,), 0, jnp.int32) + b
                for q in range(D // 16):
                    vec = buf_v[s][b, pl.ds(16 * q, 16)]
                    plsc.store_scatter(bufT_v[s], [iq[q], idx_b], vec)

        def drain_stores(s):
            for dds in range(8):
                pltpu.make_async_copy(
                    bufT_v[s].at[pl.ds(8 * dds, 8), pl.ds(0, BLK)],
                    out_hbm.at[0, 0, 0], ssems[s]).wait()

        for j in range(IBUF):
            fire_idx(j, f0 + j)
        for s in range(RBUF):
            fire_gather(s, s)

        @pl.loop(0, npw, step=IBUF)
        def _(m):
            for j in range(IBUF):
                f = f0 + m + j
                s = j % RBUF
                h, bb = hbb(f)
                pltpu.make_async_copy(table_hbm.at[idx_v[j]], buf_v[s],
                                      gsems[s]).wait()

                @pl.when(m + j + IBUF < npw)
                def _():
                    fire_idx(j, f + IBUF)

                @pl.when(m + j >= RBUF)
                def _():
                    drain_stores(s)

                transpose(s)
                for dds in range(8):
                    pltpu.async_copy(
                        bufT_v[s].at[pl.ds(8 * dds, 8), pl.ds(0, BLK)],
                        out_hbm.at[h, dds, bb], ssems[s])

                @pl.when(m + j + RBUF < npw)
                def _():
                    fire_gather((j + RBUF) % IBUF, s)

        for s in range(RBUF):
            drain_stores(s)

    fn = pl.kernel(
        body,
        out_type=jax.ShapeDtypeStruct((H, D // 8, B0 // BLK, 8, BLK),
                                      jnp.float32),
        mesh=plsc.VectorSubcoreMesh(core_axis_name="c", subcore_axis_name="s"),
        compiler_params=pltpu.CompilerParams(use_tc_tiling_on_sc=False,
                                             needs_layout_passes=False),
        scratch_types=[
            [pltpu.VMEM((BLK,), jnp.int32) for _ in range(IBUF)],
            [pltpu.VMEM((BLK, D), jnp.float32) for _ in range(RBUF)],
            [pltpu.VMEM((D, BLK + 1), jnp.float32) for _ in range(RBUF)],
            [pltpu.SemaphoreType.DMA for _ in range(IBUF)],
            [pltpu.SemaphoreType.DMA for _ in range(RBUF)],
            [pltpu.SemaphoreType.DMA for _ in range(RBUF)],
        ],
    )
    out5 = fn(xt, table)
    return out5.transpose(2, 4, 0, 1, 3).reshape(B0, H, D)
